# Initial kernel scaffold; baseline (speedup 1.0000x reference)
import jax
import jax.numpy as jnp
from jax.experimental import pallas as pl


def kernel(x, pe, edge_index, Wl0, bl0, Wr0, Wl1, bl1, Wr1, Wl2, bl2, Wr2):
    raise NotImplementedError("write your pallas kernel here")


# speedup vs baseline: 5.1069x; 5.1069x over previous
import functools

import jax
import jax.numpy as jnp
from jax import lax
from jax.experimental import pallas as pl
from jax.experimental.pallas import tpu as pltpu
from jax.experimental.pallas import tpu_sc as plsc

N = 10000
E = 160000
NPAD = 10240
NC, NS = 2, 16
NW = NC * NS
BE = 128
EPAD = 163840
NB = EPAD // (NW * BE)
RPT = NPAD // NS

MB = 256
D_X, PE, D_IN, H, OUT = 256, 16, 272, 512, 17
OUTP = 32


def _seg_sum(table, src3, dst3, zrows, nchunk, w):
    mesh = plsc.VectorSubcoreMesh(
        core_axis_name="c", subcore_axis_name="s",
        num_cores=NC, num_subcores=NS)
    out_type = jax.ShapeDtypeStruct((nchunk, NC, NPAD, w), jnp.float32)
    scratch = [
        pltpu.VMEM((NB, BE), jnp.int32),
        pltpu.VMEM((NB, BE), jnp.int32),
        pltpu.VMEM((BE, w), jnp.float32),
        pltpu.VMEM_SHARED((NPAD, w), jnp.float32),
    ]

    @functools.partial(pl.kernel, out_type=out_type, mesh=mesh,
                       scratch_types=scratch,
                       compiler_params=pltpu.CompilerParams(
                           use_tc_tiling_on_sc=(w % 128 == 0)))
    def k(table_h, src_h, dst_h, z_h, out_h, src_v, dst_v, rows_v, acc):
        core = lax.axis_index("c")
        sub = lax.axis_index("s")
        wid = sub * NC + core
        pltpu.sync_copy(src_h.at[wid], src_v)
        pltpu.sync_copy(dst_h.at[wid], dst_v)
        for c in range(nchunk):
            pltpu.sync_copy(z_h, acc.at[pl.ds(sub * RPT, RPT)])
            plsc.subcore_barrier()

            def body(j, carry):
                pltpu.sync_copy(table_h.at[c].at[src_v.at[j]], rows_v)
                pltpu.sync_copy(rows_v, acc.at[dst_v.at[j]], add=True)
                return carry

            lax.fori_loop(0, NB, body, 0)
            plsc.subcore_barrier()
            pltpu.sync_copy(acc.at[pl.ds(sub * RPT, RPT)],
                            out_h.at[c].at[core].at[pl.ds(sub * RPT, RPT)])

    return k(table, src3, dst3, zrows)


def _layer0_kernel(a128_ref, a32_ref, xp_ref, pep_ref, wl_ref, bl_ref,
                   wr_ref, h1_ref, invd_ref):
    i = pl.program_id(0)
    deg = a32_ref[0, :, 16:17] + a32_ref[1, :, 16:17]
    invd = 1.0 / jnp.maximum(deg, 1.0)
    aggx0 = a128_ref[0, 0] + a128_ref[0, 1]
    aggx1 = a128_ref[1, 0] + a128_ref[1, 1]
    aggpe = a32_ref[0, :, :16] + a32_ref[1, :, :16]
    acc = jnp.dot(aggx0, wl_ref[0:128], preferred_element_type=jnp.float32)
    acc += jnp.dot(aggx1, wl_ref[128:256], preferred_element_type=jnp.float32)
    acc += jnp.dot(aggpe, wl_ref[256:272], preferred_element_type=jnp.float32)
    acc = acc * invd + bl_ref[0]
    acc += jnp.dot(xp_ref[...], wr_ref[0:256], preferred_element_type=jnp.float32)
    acc += jnp.dot(pep_ref[...], wr_ref[256:272], preferred_element_type=jnp.float32)
    rows = i * MB + lax.broadcasted_iota(jnp.int32, (MB, 1), 0)
    h = jnp.where(rows < N, jnp.maximum(acc, 0.0), 0.0)
    h1_ref[0] = h
    invd_ref[...] = invd


def _layer0(a128, a32, xp, pep, wl0t, bl0, wr0t):
    grid = (NPAD // MB, H // 128)
    return pl.pallas_call(
        _layer0_kernel,
        grid=grid,
        in_specs=[
            pl.BlockSpec((2, 2, MB, 128), lambda i, j: (0, 0, i, 0)),
            pl.BlockSpec((2, MB, 32), lambda i, j: (0, i, 0)),
            pl.BlockSpec((MB, D_X), lambda i, j: (i, 0)),
            pl.BlockSpec((MB, PE), lambda i, j: (i, 0)),
            pl.BlockSpec((D_IN, 128), lambda i, j: (0, j)),
            pl.BlockSpec((1, 128), lambda i, j: (0, j)),
            pl.BlockSpec((D_IN, 128), lambda i, j: (0, j)),
        ],
        out_specs=[
            pl.BlockSpec((1, MB, 128), lambda i, j: (j, i, 0)),
            pl.BlockSpec((MB, 1), lambda i, j: (i, 0)),
        ],
        out_shape=[
            jax.ShapeDtypeStruct((H // 128, NPAD, 128), jnp.float32),
            jax.ShapeDtypeStruct((NPAD, 1), jnp.float32),
        ],
    )(a128, a32, xp, pep, wl0t, bl0, wr0t)


def _layer1_kernel(ah_ref, invd_ref, h1_ref, wl_ref, bl_ref, wr_ref,
                   wl2_ref, wr2_ref, bl2_ref, p2_ref, r2_ref):
    i = pl.program_id(0)
    invd = invd_ref[...]
    agg = jnp.concatenate([ah_ref[c, 0] + ah_ref[c, 1] for c in range(4)],
                          axis=1) * invd
    h1 = jnp.concatenate([h1_ref[c] for c in range(4)], axis=1)
    h2 = jnp.dot(agg, wl_ref[...], preferred_element_type=jnp.float32)
    h2 += bl_ref[0]
    h2 += jnp.dot(h1, wr_ref[...], preferred_element_type=jnp.float32)
    rows = i * MB + lax.broadcasted_iota(jnp.int32, (MB, 1), 0)
    h2 = jnp.where(rows < N, jnp.maximum(h2, 0.0), 0.0)
    p2_ref[...] = jnp.dot(h2, wl2_ref[...], preferred_element_type=jnp.float32)
    r2_ref[...] = jnp.dot(h2, wr2_ref[...],
                          preferred_element_type=jnp.float32) + bl2_ref[0]


def _layer1(ah, invd, h1c, wl1t, bl1, wr1t, wl2pt, wr2pt, bl2p):
    grid = (NPAD // MB,)
    return pl.pallas_call(
        _layer1_kernel,
        grid=grid,
        in_specs=[
            pl.BlockSpec((4, 2, MB, 128), lambda i: (0, 0, i, 0)),
            pl.BlockSpec((MB, 1), lambda i: (i, 0)),
            pl.BlockSpec((4, MB, 128), lambda i: (0, i, 0)),
            pl.BlockSpec((H, H), lambda i: (0, 0)),
            pl.BlockSpec((1, H), lambda i: (0, 0)),
            pl.BlockSpec((H, H), lambda i: (0, 0)),
            pl.BlockSpec((H, OUTP), lambda i: (0, 0)),
            pl.BlockSpec((H, OUTP), lambda i: (0, 0)),
            pl.BlockSpec((1, OUTP), lambda i: (0, 0)),
        ],
        out_specs=[
            pl.BlockSpec((MB, OUTP), lambda i: (i, 0)),
            pl.BlockSpec((MB, OUTP), lambda i: (i, 0)),
        ],
        out_shape=[
            jax.ShapeDtypeStruct((NPAD, OUTP), jnp.float32),
            jax.ShapeDtypeStruct((NPAD, OUTP), jnp.float32),
        ],
    )(ah, invd, h1c, wl1t, bl1, wr1t, wl2pt, wr2pt, bl2p)


MBC = 400


def _head_kernel(ap_ref, invd_ref, r2_ref, o1_ref, o2_ref):
    maps = (ap_ref[0] + ap_ref[1]) * invd_ref[...] + r2_ref[...]
    o1_ref[...] = jnp.tanh(maps[:, :16])
    o2_ref[...] = jnp.minimum(jnp.exp(maps[:, 16:17]), 10.0)


def _head(ap, invd, r2):
    grid = (N // MBC,)
    return pl.pallas_call(
        _head_kernel,
        grid=grid,
        in_specs=[
            pl.BlockSpec((2, MBC, OUTP), lambda i: (0, i, 0)),
            pl.BlockSpec((MBC, 1), lambda i: (i, 0)),
            pl.BlockSpec((MBC, OUTP), lambda i: (i, 0)),
        ],
        out_specs=[
            pl.BlockSpec((MBC, 16), lambda i: (i, 0)),
            pl.BlockSpec((MBC, 1), lambda i: (i, 0)),
        ],
        out_shape=[
            jax.ShapeDtypeStruct((N, 16), jnp.float32),
            jax.ShapeDtypeStruct((N, 1), jnp.float32),
        ],
    )(ap, invd, r2)


def kernel(x, pe, edge_index, Wl0, bl0, Wr0, Wl1, bl1, Wr1, Wl2, bl2, Wr2):
    f32 = jnp.float32
    src = edge_index[0]
    dst = edge_index[1]
    npe = EPAD - E
    pad_src = N + (jnp.arange(npe, dtype=jnp.int32) % (NPAD - N))
    pad_dst = jnp.arange(npe, dtype=jnp.int32) % N
    src3 = jnp.concatenate([src, pad_src]).reshape(NW, NB, BE)
    dst3 = jnp.concatenate([dst, pad_dst]).reshape(NW, NB, BE)

    xp = jnp.zeros((NPAD, D_X), f32).at[:N].set(x)
    pep = jnp.zeros((NPAD, PE), f32).at[:N].set(pe)
    t128 = jnp.stack([xp[:, :128], xp[:, 128:]])
    ones_col = (jnp.arange(NPAD) < N).astype(f32)[:, None]
    t32 = jnp.concatenate([pep, ones_col, jnp.zeros((NPAD, 15), f32)],
                          axis=1)[None]
    z128 = jnp.zeros((RPT, 128), f32)
    z32 = jnp.zeros((RPT, 32), f32)

    wl0t, wr0t = Wl0.T, Wr0.T
    wl1t, wr1t = Wl1.T, Wr1.T
    wl2pt = jnp.zeros((H, OUTP), f32).at[:, :OUT].set(Wl2.T)
    wr2pt = jnp.zeros((H, OUTP), f32).at[:, :OUT].set(Wr2.T)
    bl2p = jnp.zeros((1, OUTP), f32).at[0, :OUT].set(bl2)

    a128 = _seg_sum(t128, src3, dst3, z128, 2, 128)
    a32 = _seg_sum(t32, src3, dst3, z32, 1, 32)[0]
    h1c, invd = _layer0(a128, a32, xp, pep, wl0t, bl0[None], wr0t)

    ah = _seg_sum(h1c, src3, dst3, z128, 4, 128)
    p2, r2 = _layer1(ah, invd, h1c, wl1t, bl1[None], wr1t, wl2pt, wr2pt, bl2p)

    ap = _seg_sum(p2[None], src3, dst3, z32, 1, 32)[0]
    o1, o2 = _head(ap, invd, r2)
    return (o1, o2[:, 0])


# speedup vs baseline: 6.8685x; 1.3449x over previous
import functools

import jax
import jax.numpy as jnp
from jax import lax
from jax.experimental import pallas as pl
from jax.experimental.pallas import tpu as pltpu
from jax.experimental.pallas import tpu_sc as plsc

N = 10000
E = 160000
NPAD = 10240
NC, NS = 2, 16
NW = NC * NS
BE = 128
EPAD = 163840
NB = EPAD // (NW * BE)
RPT = NPAD // NS

MB = 256
D_X, PE, D_IN, H, OUT = 256, 16, 272, 512, 17
OUTP = 32


def _seg_sum(table, src3, dst3, zrows, nchunk, w):
    mesh = plsc.VectorSubcoreMesh(
        core_axis_name="c", subcore_axis_name="s",
        num_cores=NC, num_subcores=NS)
    out_type = jax.ShapeDtypeStruct((nchunk, NC, NPAD, w), jnp.float32)
    scratch = [
        pltpu.VMEM((NB, BE), jnp.int32),
        pltpu.VMEM((NB, BE), jnp.int32),
        pltpu.VMEM((BE, w), jnp.float32),
        pltpu.VMEM((BE, w), jnp.float32),
        pltpu.VMEM_SHARED((NPAD, w), jnp.float32),
        pltpu.SemaphoreType.DMA,
        pltpu.SemaphoreType.DMA,
    ]

    @functools.partial(pl.kernel, out_type=out_type, mesh=mesh,
                       scratch_types=scratch,
                       compiler_params=pltpu.CompilerParams(
                           use_tc_tiling_on_sc=(w % 128 == 0)))
    def k(table_h, src_h, dst_h, z_h, out_h, src_v, dst_v, rows0, rows1,
          acc, sem0, sem1):
        core = lax.axis_index("c")
        sub = lax.axis_index("s")
        wid = sub * NC + core
        pltpu.sync_copy(src_h.at[wid], src_v)
        pltpu.sync_copy(dst_h.at[wid], dst_v)
        for c in range(nchunk):
            tab = table_h.at[c]
            pltpu.sync_copy(z_h, acc.at[pl.ds(sub * RPT, RPT)])
            plsc.subcore_barrier()
            pltpu.async_copy(tab.at[src_v.at[0]], rows0, sem0)

            def body(jj, carry):
                j = jj * 2
                pltpu.async_copy(tab.at[src_v.at[j + 1]], rows1, sem1)
                pltpu.make_async_copy(tab.at[src_v.at[j]], rows0, sem0).wait()
                pltpu.sync_copy(rows0, acc.at[dst_v.at[j]], add=True)

                @pl.when(j + 2 < NB)
                def _():
                    pltpu.async_copy(tab.at[src_v.at[j + 2]], rows0, sem0)

                pltpu.make_async_copy(tab.at[src_v.at[j + 1]], rows1,
                                      sem1).wait()
                pltpu.sync_copy(rows1, acc.at[dst_v.at[j + 1]], add=True)
                return carry

            lax.fori_loop(0, NB // 2, body, 0)
            plsc.subcore_barrier()
            pltpu.sync_copy(acc.at[pl.ds(sub * RPT, RPT)],
                            out_h.at[c].at[core].at[pl.ds(sub * RPT, RPT)])

    return k(table, src3, dst3, zrows)


def _layer0_kernel(a128_ref, a32_ref, xp_ref, pep_ref, wl_ref, bl_ref,
                   wr_ref, h1_ref, invd_ref):
    i = pl.program_id(0)
    deg = a32_ref[0, :, 16:17] + a32_ref[1, :, 16:17]
    invd = 1.0 / jnp.maximum(deg, 1.0)
    aggx0 = a128_ref[0, 0] + a128_ref[0, 1]
    aggx1 = a128_ref[1, 0] + a128_ref[1, 1]
    aggpe = a32_ref[0, :, :16] + a32_ref[1, :, :16]
    acc = jnp.dot(aggx0, wl_ref[0:128], preferred_element_type=jnp.float32)
    acc += jnp.dot(aggx1, wl_ref[128:256], preferred_element_type=jnp.float32)
    acc += jnp.dot(aggpe, wl_ref[256:272], preferred_element_type=jnp.float32)
    acc = acc * invd + bl_ref[0]
    acc += jnp.dot(xp_ref[...], wr_ref[0:256], preferred_element_type=jnp.float32)
    acc += jnp.dot(pep_ref[...], wr_ref[256:272], preferred_element_type=jnp.float32)
    rows = i * MB + lax.broadcasted_iota(jnp.int32, (MB, 1), 0)
    h = jnp.where(rows < N, jnp.maximum(acc, 0.0), 0.0)
    h1_ref[0] = h
    invd_ref[...] = invd


def _layer0(a128, a32, xp, pep, wl0t, bl0, wr0t):
    grid = (NPAD // MB, H // 128)
    return pl.pallas_call(
        _layer0_kernel,
        grid=grid,
        in_specs=[
            pl.BlockSpec((2, 2, MB, 128), lambda i, j: (0, 0, i, 0)),
            pl.BlockSpec((2, MB, 32), lambda i, j: (0, i, 0)),
            pl.BlockSpec((MB, D_X), lambda i, j: (i, 0)),
            pl.BlockSpec((MB, PE), lambda i, j: (i, 0)),
            pl.BlockSpec((D_IN, 128), lambda i, j: (0, j)),
            pl.BlockSpec((1, 128), lambda i, j: (0, j)),
            pl.BlockSpec((D_IN, 128), lambda i, j: (0, j)),
        ],
        out_specs=[
            pl.BlockSpec((1, MB, 128), lambda i, j: (j, i, 0)),
            pl.BlockSpec((MB, 1), lambda i, j: (i, 0)),
        ],
        out_shape=[
            jax.ShapeDtypeStruct((H // 128, NPAD, 128), jnp.float32),
            jax.ShapeDtypeStruct((NPAD, 1), jnp.float32),
        ],
    )(a128, a32, xp, pep, wl0t, bl0, wr0t)


def _layer1_kernel(ah_ref, invd_ref, h1_ref, wl_ref, bl_ref, wr_ref,
                   wl2_ref, wr2_ref, bl2_ref, p2_ref, r2_ref):
    i = pl.program_id(0)
    invd = invd_ref[...]
    agg = jnp.concatenate([ah_ref[c, 0] + ah_ref[c, 1] for c in range(4)],
                          axis=1) * invd
    h1 = jnp.concatenate([h1_ref[c] for c in range(4)], axis=1)
    h2 = jnp.dot(agg, wl_ref[...], preferred_element_type=jnp.float32)
    h2 += bl_ref[0]
    h2 += jnp.dot(h1, wr_ref[...], preferred_element_type=jnp.float32)
    rows = i * MB + lax.broadcasted_iota(jnp.int32, (MB, 1), 0)
    h2 = jnp.where(rows < N, jnp.maximum(h2, 0.0), 0.0)
    p2_ref[...] = jnp.dot(h2, wl2_ref[...], preferred_element_type=jnp.float32)
    r2_ref[...] = jnp.dot(h2, wr2_ref[...],
                          preferred_element_type=jnp.float32) + bl2_ref[0]


def _layer1(ah, invd, h1c, wl1t, bl1, wr1t, wl2pt, wr2pt, bl2p):
    grid = (NPAD // MB,)
    return pl.pallas_call(
        _layer1_kernel,
        grid=grid,
        in_specs=[
            pl.BlockSpec((4, 2, MB, 128), lambda i: (0, 0, i, 0)),
            pl.BlockSpec((MB, 1), lambda i: (i, 0)),
            pl.BlockSpec((4, MB, 128), lambda i: (0, i, 0)),
            pl.BlockSpec((H, H), lambda i: (0, 0)),
            pl.BlockSpec((1, H), lambda i: (0, 0)),
            pl.BlockSpec((H, H), lambda i: (0, 0)),
            pl.BlockSpec((H, OUTP), lambda i: (0, 0)),
            pl.BlockSpec((H, OUTP), lambda i: (0, 0)),
            pl.BlockSpec((1, OUTP), lambda i: (0, 0)),
        ],
        out_specs=[
            pl.BlockSpec((MB, OUTP), lambda i: (i, 0)),
            pl.BlockSpec((MB, OUTP), lambda i: (i, 0)),
        ],
        out_shape=[
            jax.ShapeDtypeStruct((NPAD, OUTP), jnp.float32),
            jax.ShapeDtypeStruct((NPAD, OUTP), jnp.float32),
        ],
    )(ah, invd, h1c, wl1t, bl1, wr1t, wl2pt, wr2pt, bl2p)


MBC = 400


def _head_kernel(ap_ref, invd_ref, r2_ref, o1_ref, o2_ref):
    maps = (ap_ref[0] + ap_ref[1]) * invd_ref[...] + r2_ref[...]
    o1_ref[...] = jnp.tanh(maps[:, :16])
    o2_ref[...] = jnp.minimum(jnp.exp(maps[:, 16:17]), 10.0)


def _head(ap, invd, r2):
    grid = (N // MBC,)
    return pl.pallas_call(
        _head_kernel,
        grid=grid,
        in_specs=[
            pl.BlockSpec((2, MBC, OUTP), lambda i: (0, i, 0)),
            pl.BlockSpec((MBC, 1), lambda i: (i, 0)),
            pl.BlockSpec((MBC, OUTP), lambda i: (i, 0)),
        ],
        out_specs=[
            pl.BlockSpec((MBC, 16), lambda i: (i, 0)),
            pl.BlockSpec((MBC, 1), lambda i: (i, 0)),
        ],
        out_shape=[
            jax.ShapeDtypeStruct((N, 16), jnp.float32),
            jax.ShapeDtypeStruct((N, 1), jnp.float32),
        ],
    )(ap, invd, r2)


def kernel(x, pe, edge_index, Wl0, bl0, Wr0, Wl1, bl1, Wr1, Wl2, bl2, Wr2):
    f32 = jnp.float32
    src = edge_index[0]
    dst = edge_index[1]
    npe = EPAD - E
    pad_src = N + (jnp.arange(npe, dtype=jnp.int32) % (NPAD - N))
    pad_dst = jnp.arange(npe, dtype=jnp.int32) % N
    src3 = jnp.concatenate([src, pad_src]).reshape(NW, NB, BE)
    dst3 = jnp.concatenate([dst, pad_dst]).reshape(NW, NB, BE)

    xp = jnp.zeros((NPAD, D_X), f32).at[:N].set(x)
    pep = jnp.zeros((NPAD, PE), f32).at[:N].set(pe)
    t128 = jnp.stack([xp[:, :128], xp[:, 128:]])
    ones_col = (jnp.arange(NPAD) < N).astype(f32)[:, None]
    t32 = jnp.concatenate([pep, ones_col, jnp.zeros((NPAD, 15), f32)],
                          axis=1)[None]
    z128 = jnp.zeros((RPT, 128), f32)
    z32 = jnp.zeros((RPT, 32), f32)

    wl0t, wr0t = Wl0.T, Wr0.T
    wl1t, wr1t = Wl1.T, Wr1.T
    wl2pt = jnp.zeros((H, OUTP), f32).at[:, :OUT].set(Wl2.T)
    wr2pt = jnp.zeros((H, OUTP), f32).at[:, :OUT].set(Wr2.T)
    bl2p = jnp.zeros((1, OUTP), f32).at[0, :OUT].set(bl2)

    a128 = _seg_sum(t128, src3, dst3, z128, 2, 128)
    a32 = _seg_sum(t32, src3, dst3, z32, 1, 32)[0]
    h1c, invd = _layer0(a128, a32, xp, pep, wl0t, bl0[None], wr0t)

    ah = _seg_sum(h1c, src3, dst3, z128, 4, 128)
    p2, r2 = _layer1(ah, invd, h1c, wl1t, bl1[None], wr1t, wl2pt, wr2pt, bl2p)

    ap = _seg_sum(p2[None], src3, dst3, z32, 1, 32)[0]
    o1, o2 = _head(ap, invd, r2)
    return (o1, o2[:, 0])


# speedup vs baseline: 7.0441x; 1.0256x over previous
import functools

import jax
import jax.numpy as jnp
from jax import lax
from jax.experimental import pallas as pl
from jax.experimental.pallas import tpu as pltpu
from jax.experimental.pallas import tpu_sc as plsc

N = 10000
E = 160000
NPAD = 10240
NC, NS = 2, 16
NW = NC * NS
BE = 128
EPAD = 163840
NB = EPAD // (NW * BE)
RPT = NPAD // NS

MB = 256
D_X, PE, D_IN, H, OUT = 256, 16, 272, 512, 17
OUTP = 32


def _seg_sum(table, src3, dst3, zrows, nchunk, w):
    mesh = plsc.VectorSubcoreMesh(
        core_axis_name="c", subcore_axis_name="s",
        num_cores=NC, num_subcores=NS)
    out_type = jax.ShapeDtypeStruct((nchunk, NC, NPAD, w), jnp.float32)
    scratch = [
        pltpu.VMEM((NB, BE), jnp.int32),
        pltpu.VMEM((NB, BE), jnp.int32),
        pltpu.VMEM((BE, w), jnp.float32),
        pltpu.VMEM((BE, w), jnp.float32),
        pltpu.VMEM_SHARED((NPAD, w), jnp.float32),
        pltpu.SemaphoreType.DMA,
        pltpu.SemaphoreType.DMA,
    ]

    @functools.partial(pl.kernel, out_type=out_type, mesh=mesh,
                       scratch_types=scratch,
                       compiler_params=pltpu.CompilerParams(
                           use_tc_tiling_on_sc=(w % 128 == 0)))
    def k(table_h, src_h, dst_h, z_h, out_h, src_v, dst_v, rows0, rows1,
          acc, sem0, sem1):
        core = lax.axis_index("c")
        sub = lax.axis_index("s")
        wid = sub * NC + core
        pltpu.sync_copy(src_h.at[wid], src_v)
        pltpu.sync_copy(dst_h.at[wid], dst_v)
        for c in range(nchunk):
            tab = table_h.at[c]
            pltpu.sync_copy(z_h, acc.at[pl.ds(sub * RPT, RPT)])
            plsc.subcore_barrier()
            pltpu.async_copy(tab.at[src_v.at[0]], rows0, sem0)

            def body(jj, carry):
                j = jj * 2
                pltpu.async_copy(tab.at[src_v.at[j + 1]], rows1, sem1)
                pltpu.make_async_copy(tab.at[src_v.at[j]], rows0, sem0).wait()
                pltpu.sync_copy(rows0, acc.at[dst_v.at[j]], add=True)

                @pl.when(j + 2 < NB)
                def _():
                    pltpu.async_copy(tab.at[src_v.at[j + 2]], rows0, sem0)

                pltpu.make_async_copy(tab.at[src_v.at[j + 1]], rows1,
                                      sem1).wait()
                pltpu.sync_copy(rows1, acc.at[dst_v.at[j + 1]], add=True)
                return carry

            lax.fori_loop(0, NB // 2, body, 0)
            plsc.subcore_barrier()
            pltpu.sync_copy(acc.at[pl.ds(sub * RPT, RPT)],
                            out_h.at[c].at[core].at[pl.ds(sub * RPT, RPT)])

    return k(table, src3, dst3, zrows)


def _layer0_kernel(a128_ref, a32_ref, xp_ref, pep_ref, wl_ref, bl_ref,
                   wr_ref, h1_ref, invd_ref):
    i = pl.program_id(0)
    bf = jnp.bfloat16
    deg = a32_ref[0, :, 16:17] + a32_ref[1, :, 16:17]
    invd = 1.0 / jnp.maximum(deg, 1.0)
    aggx0 = (a128_ref[0, 0] + a128_ref[0, 1]).astype(bf)
    aggx1 = (a128_ref[1, 0] + a128_ref[1, 1]).astype(bf)
    aggpe = (a32_ref[0, :, :16] + a32_ref[1, :, :16]).astype(bf)
    acc = jnp.dot(aggx0, wl_ref[0:128], preferred_element_type=jnp.float32)
    acc += jnp.dot(aggx1, wl_ref[128:256], preferred_element_type=jnp.float32)
    acc += jnp.dot(aggpe, wl_ref[256:272], preferred_element_type=jnp.float32)
    acc = acc * invd + bl_ref[0]
    acc += jnp.dot(xp_ref[...], wr_ref[0:256], preferred_element_type=jnp.float32)
    acc += jnp.dot(pep_ref[...], wr_ref[256:272], preferred_element_type=jnp.float32)
    rows = i * MB + lax.broadcasted_iota(jnp.int32, (MB, 1), 0)
    h = jnp.where(rows < N, jnp.maximum(acc, 0.0), 0.0)
    h1_ref[0] = h
    invd_ref[...] = invd


def _layer0(a128, a32, xp, pep, wl0t, bl0, wr0t):
    grid = (NPAD // MB, H // 128)
    return pl.pallas_call(
        _layer0_kernel,
        grid=grid,
        in_specs=[
            pl.BlockSpec((2, 2, MB, 128), lambda i, j: (0, 0, i, 0)),
            pl.BlockSpec((2, MB, 32), lambda i, j: (0, i, 0)),
            pl.BlockSpec((MB, D_X), lambda i, j: (i, 0)),
            pl.BlockSpec((MB, PE), lambda i, j: (i, 0)),
            pl.BlockSpec((D_IN, 128), lambda i, j: (0, j)),
            pl.BlockSpec((1, 128), lambda i, j: (0, j)),
            pl.BlockSpec((D_IN, 128), lambda i, j: (0, j)),
        ],
        out_specs=[
            pl.BlockSpec((1, MB, 128), lambda i, j: (j, i, 0)),
            pl.BlockSpec((MB, 1), lambda i, j: (i, 0)),
        ],
        out_shape=[
            jax.ShapeDtypeStruct((H // 128, NPAD, 128), jnp.float32),
            jax.ShapeDtypeStruct((NPAD, 1), jnp.float32),
        ],
    )(a128, a32, xp, pep, wl0t, bl0, wr0t)


def _layer1_kernel(ah_ref, invd_ref, h1_ref, wl_ref, bl_ref, wr_ref,
                   wl2_ref, wr2_ref, bl2_ref, p2_ref, r2_ref):
    i = pl.program_id(0)
    bf = jnp.bfloat16
    invd = invd_ref[...]
    agg = jnp.concatenate([ah_ref[c, 0] + ah_ref[c, 1] for c in range(4)],
                          axis=1) * invd
    h1 = jnp.concatenate([h1_ref[c] for c in range(4)], axis=1)
    h2 = jnp.dot(agg.astype(bf), wl_ref[...], preferred_element_type=jnp.float32)
    h2 += bl_ref[0]
    h2 += jnp.dot(h1.astype(bf), wr_ref[...], preferred_element_type=jnp.float32)
    rows = i * MB + lax.broadcasted_iota(jnp.int32, (MB, 1), 0)
    h2 = jnp.where(rows < N, jnp.maximum(h2, 0.0), 0.0).astype(bf)
    p2_ref[...] = jnp.dot(h2, wl2_ref[...], preferred_element_type=jnp.float32)
    r2_ref[...] = jnp.dot(h2, wr2_ref[...],
                          preferred_element_type=jnp.float32) + bl2_ref[0]


def _layer1(ah, invd, h1c, wl1t, bl1, wr1t, wl2pt, wr2pt, bl2p):
    grid = (NPAD // MB,)
    return pl.pallas_call(
        _layer1_kernel,
        grid=grid,
        in_specs=[
            pl.BlockSpec((4, 2, MB, 128), lambda i: (0, 0, i, 0)),
            pl.BlockSpec((MB, 1), lambda i: (i, 0)),
            pl.BlockSpec((4, MB, 128), lambda i: (0, i, 0)),
            pl.BlockSpec((H, H), lambda i: (0, 0)),
            pl.BlockSpec((1, H), lambda i: (0, 0)),
            pl.BlockSpec((H, H), lambda i: (0, 0)),
            pl.BlockSpec((H, OUTP), lambda i: (0, 0)),
            pl.BlockSpec((H, OUTP), lambda i: (0, 0)),
            pl.BlockSpec((1, OUTP), lambda i: (0, 0)),
        ],
        out_specs=[
            pl.BlockSpec((MB, OUTP), lambda i: (i, 0)),
            pl.BlockSpec((MB, OUTP), lambda i: (i, 0)),
        ],
        out_shape=[
            jax.ShapeDtypeStruct((NPAD, OUTP), jnp.float32),
            jax.ShapeDtypeStruct((NPAD, OUTP), jnp.float32),
        ],
    )(ah, invd, h1c, wl1t, bl1, wr1t, wl2pt, wr2pt, bl2p)


MBC = 400


def _head_kernel(ap_ref, invd_ref, r2_ref, o1_ref, o2_ref):
    maps = (ap_ref[0] + ap_ref[1]) * invd_ref[...] + r2_ref[...]
    o1_ref[...] = jnp.tanh(maps[:, :16])
    o2_ref[...] = jnp.minimum(jnp.exp(maps[:, 16:17]), 10.0)


def _head(ap, invd, r2):
    grid = (N // MBC,)
    return pl.pallas_call(
        _head_kernel,
        grid=grid,
        in_specs=[
            pl.BlockSpec((2, MBC, OUTP), lambda i: (0, i, 0)),
            pl.BlockSpec((MBC, 1), lambda i: (i, 0)),
            pl.BlockSpec((MBC, OUTP), lambda i: (i, 0)),
        ],
        out_specs=[
            pl.BlockSpec((MBC, 16), lambda i: (i, 0)),
            pl.BlockSpec((MBC, 1), lambda i: (i, 0)),
        ],
        out_shape=[
            jax.ShapeDtypeStruct((N, 16), jnp.float32),
            jax.ShapeDtypeStruct((N, 1), jnp.float32),
        ],
    )(ap, invd, r2)


def kernel(x, pe, edge_index, Wl0, bl0, Wr0, Wl1, bl1, Wr1, Wl2, bl2, Wr2):
    f32 = jnp.float32
    src = edge_index[0]
    dst = edge_index[1]
    npe = EPAD - E
    pad_src = N + (jnp.arange(npe, dtype=jnp.int32) % (NPAD - N))
    pad_dst = jnp.arange(npe, dtype=jnp.int32) % N
    src3 = jnp.concatenate([src, pad_src]).reshape(NW, NB, BE)
    dst3 = jnp.concatenate([dst, pad_dst]).reshape(NW, NB, BE)

    xp = jnp.zeros((NPAD, D_X), f32).at[:N].set(x)
    pep = jnp.zeros((NPAD, PE), f32).at[:N].set(pe)
    t128 = jnp.stack([xp[:, :128], xp[:, 128:]])
    ones_col = (jnp.arange(NPAD) < N).astype(f32)[:, None]
    t32 = jnp.concatenate([pep, ones_col, jnp.zeros((NPAD, 15), f32)],
                          axis=1)[None]
    z128 = jnp.zeros((RPT, 128), f32)
    z32 = jnp.zeros((RPT, 32), f32)

    bf = jnp.bfloat16
    wl0t, wr0t = Wl0.T.astype(bf), Wr0.T.astype(bf)
    wl1t, wr1t = Wl1.T.astype(bf), Wr1.T.astype(bf)
    wl2pt = jnp.zeros((H, OUTP), f32).at[:, :OUT].set(Wl2.T).astype(bf)
    wr2pt = jnp.zeros((H, OUTP), f32).at[:, :OUT].set(Wr2.T).astype(bf)
    bl2p = jnp.zeros((1, OUTP), f32).at[0, :OUT].set(bl2)
    xpb, pepb = xp.astype(bf), pep.astype(bf)

    a128 = _seg_sum(t128, src3, dst3, z128, 2, 128)
    a32 = _seg_sum(t32, src3, dst3, z32, 1, 32)[0]
    h1c, invd = _layer0(a128, a32, xpb, pepb, wl0t, bl0[None], wr0t)

    ah = _seg_sum(h1c, src3, dst3, z128, 4, 128)
    p2, r2 = _layer1(ah, invd, h1c, wl1t, bl1[None], wr1t, wl2pt, wr2pt, bl2p)

    ap = _seg_sum(p2[None], src3, dst3, z32, 1, 32)[0]
    o1, o2 = _head(ap, invd, r2)
    return (o1, o2[:, 0])
